# Initial kernel scaffold; baseline (speedup 1.0000x reference)
import jax
import jax.numpy as jnp
from jax.experimental import pallas as pl


def kernel(x, edge_index, W_gat, att_src, att_dst, b_gat, W_emb, b_emb, W1, b1, W2, b2):
    raise NotImplementedError("write your pallas kernel here")


# speedup vs baseline: 8.6808x; 8.6808x over previous
import functools

import jax
import jax.numpy as jnp
from jax import lax
from jax.experimental import pallas as pl
from jax.experimental.pallas import tpu as pltpu
from jax.experimental.pallas import tpu_sc as plsc

N_NODES = 10000
E_RAW = 320000
E_TOT = E_RAW + N_NODES
NC, NS = 2, 16
NW = NC * NS
CHUNK = 128
EPAD = ((E_TOT + NW * CHUNK - 1) // (NW * CHUNK)) * (NW * CHUNK)
PER_W = EPAD // NW
N_CHUNKS = PER_W // CHUNK
EBLK = 2048
NBLK = 2000

_mesh = plsc.VectorSubcoreMesh(core_axis_name="c", subcore_axis_name="s",
                               num_cores=NC)


def _sc_gather(d):
    @functools.partial(
        pl.kernel, mesh=_mesh,
        out_type=jax.ShapeDtypeStruct((EPAD, d), jnp.float32),
        scratch_types=[
            pltpu.VMEM((CHUNK,), jnp.int32),
            pltpu.VMEM((CHUNK, d), jnp.float32),
            pltpu.SemaphoreType.DMA,
        ],
    )
    def k(table_hbm, idx_hbm, out_hbm, idx_v, rows_v, sem):
        wid = lax.axis_index("s") * NC + lax.axis_index("c")

        def chunk(i, carry):
            b = wid * PER_W + i * CHUNK
            pltpu.sync_copy(idx_hbm.at[pl.ds(b, CHUNK)], idx_v)
            pltpu.async_copy(table_hbm.at[idx_v], rows_v, sem).wait()
            pltpu.sync_copy(rows_v, out_hbm.at[pl.ds(b, CHUNK)])
            return carry

        lax.fori_loop(0, N_CHUNKS, chunk, 0)

    return k


def _sc_scatter_add(d):
    @functools.partial(
        pl.kernel, mesh=_mesh,
        out_type=jax.ShapeDtypeStruct((NC, N_NODES, d), jnp.float32),
        scratch_types=[
            pltpu.VMEM((CHUNK,), jnp.int32),
            pltpu.VMEM((CHUNK, d), jnp.float32),
            pltpu.VMEM_SHARED((N_NODES, d), jnp.float32),
        ],
    )
    def k(rows_hbm, idx_hbm, zeros_hbm, out_hbm, idx_v, rows_v, accum):
        c = lax.axis_index("c")
        s = lax.axis_index("s")

        @pl.when(s == 0)
        def _():
            pltpu.sync_copy(zeros_hbm, accum)

        plsc.subcore_barrier()

        def chunk(i, carry):
            b = (s * NC + c) * PER_W + i * CHUNK
            pltpu.sync_copy(idx_hbm.at[pl.ds(b, CHUNK)], idx_v)
            pltpu.sync_copy(rows_hbm.at[pl.ds(b, CHUNK)], rows_v)
            pltpu.sync_copy(rows_v, accum.at[idx_v], add=True)
            return carry

        lax.fori_loop(0, N_CHUNKS, chunk, 0)
        plsc.subcore_barrier()

        @pl.when(s == 0)
        def _():
            pltpu.sync_copy(accum, out_hbm.at[c])

    return k


def _tc1_body(x_ref, wg_ref, asrc_ref, adst_ref, h_ref, asd_ref):
    h = jnp.dot(x_ref[...], wg_ref[...],
                preferred_element_type=jnp.float32,
                precision=lax.Precision.HIGHEST)
    h_ref[...] = h
    h3 = h.reshape(h.shape[0], 8, 64)
    a_s = (h3 * asrc_ref[...][None]).sum(-1)
    a_d = (h3 * adst_ref[...][None]).sum(-1)
    asd_ref[...] = jnp.concatenate(
        [a_s, a_d, jnp.zeros((NBLK, 112), jnp.float32)], axis=1)


def _valid_col(pid, blk):
    row = pid * blk + lax.broadcasted_iota(jnp.int32, (blk, 1), 0)
    return (row < E_TOT).astype(jnp.float32)


def _tc2_body(gs_ref, gd_ref, ex_ref):
    valid = _valid_col(pl.program_id(0), EBLK)
    e = gs_ref[...][:, 0:8] + gd_ref[...][:, 8:16]
    e = jnp.where(e >= 0, e, 0.2 * e)
    ex = jnp.exp(e) * valid
    ex_ref[...] = jnp.concatenate(
        [ex, valid, jnp.zeros((EBLK, 119), jnp.float32)], axis=1)


def _tc4_body(dd_ref, q_ref):
    dd = dd_ref[...][0] + dd_ref[...][1]
    den = dd[:, 0:8]
    deg = dd[:, 8:9]
    dinv = jnp.where(deg > 0, lax.rsqrt(jnp.maximum(deg, 1e-12)), 0.0)
    q_ref[...] = jnp.concatenate(
        [den, dinv, jnp.zeros((NBLK, 119), jnp.float32)], axis=1)


def _tc5_body(ex_ref, qs_ref, qd_ref, al_ref):
    valid = _valid_col(pl.program_id(0), EBLK)
    alpha = ex_ref[...][:, 0:8] / qd_ref[...][:, 0:8]
    norm = qs_ref[...][:, 8:9] * qd_ref[...][:, 8:9] * valid
    al_ref[...] = jnp.concatenate(
        [alpha, norm, jnp.zeros((EBLK, 7), jnp.float32)], axis=1)


def _tc6_body(hs_ref, al_ref, m0_ref, m1_ref, m2_ref, m3_ref):
    a3 = al_ref[...][:, 0:8].reshape(EBLK, 8, 1)
    h3 = hs_ref[...].reshape(EBLK, 8, 64)
    msg = (h3 * a3).reshape(EBLK, 512)
    m0_ref[...] = msg[:, 0:128]
    m1_ref[...] = msg[:, 128:256]
    m2_ref[...] = msg[:, 256:384]
    m3_ref[...] = msg[:, 384:512]


def _elu(x):
    return jnp.where(x > 0, x, jnp.exp(x) - 1.0)


def _tc7_body(g0_ref, g1_ref, g2_ref, g3_ref, bg_ref, we_ref, be_ref,
              w1_ref, xw1_ref):
    outg = jnp.concatenate(
        [g0_ref[...][0] + g0_ref[...][1],
         g1_ref[...][0] + g1_ref[...][1],
         g2_ref[...][0] + g2_ref[...][1],
         g3_ref[...][0] + g3_ref[...][1]], axis=1)
    g = _elu(outg + bg_ref[...])
    g = _elu(jnp.dot(g, we_ref[...], preferred_element_type=jnp.float32,
                     precision=lax.Precision.HIGHEST) + be_ref[...])
    xw1 = jnp.dot(g, w1_ref[...], preferred_element_type=jnp.float32,
                  precision=lax.Precision.HIGHEST)
    xw1_ref[...] = jnp.concatenate(
        [xw1, jnp.zeros((NBLK, 64), jnp.float32)], axis=1)


def _tc8_body(xs_ref, al_ref, m_ref):
    m_ref[...] = xs_ref[...] * al_ref[...][:, 8:9]


def _tc9_body(p_ref, b1_ref, w2_ref, xw2_ref):
    p = p_ref[...][0] + p_ref[...][1]
    g = _elu(p[:, 0:64] + b1_ref[...])
    xw2_ref[...] = jnp.dot(g, w2_ref[...], preferred_element_type=jnp.float32,
                           precision=lax.Precision.HIGHEST)


def _tc11_body(p_ref, b2_ref, out_ref):
    out_ref[...] = p_ref[...][0] + p_ref[...][1] + b2_ref[...]


def _node_spec(d):
    return pl.BlockSpec((NBLK, d), lambda i: (i, 0))


def _edge_spec(d):
    return pl.BlockSpec((EBLK, d), lambda i: (i, 0))


def _full_spec(shape):
    nd = len(shape)
    return pl.BlockSpec(shape, lambda i: (0,) * nd)


def _part_spec(d):
    return pl.BlockSpec((NC, NBLK, d), lambda i: (0, i, 0))


def kernel(x, edge_index, W_gat, att_src, att_dst, b_gat, W_emb, b_emb,
           W1, b1, W2, b2):
    n = N_NODES
    loops = jnp.arange(n, dtype=jnp.int32)
    pad = jnp.zeros((EPAD - E_TOT,), jnp.int32)
    src = jnp.concatenate([edge_index[0].astype(jnp.int32), loops, pad])
    dst = jnp.concatenate([edge_index[1].astype(jnp.int32), loops, pad])

    ngrid = n // NBLK
    egrid = EPAD // EBLK

    h, asd = pl.pallas_call(
        _tc1_body,
        grid=(ngrid,),
        in_specs=[_node_spec(128), _full_spec((128, 512)),
                  _full_spec((8, 64)), _full_spec((8, 64))],
        out_specs=[_node_spec(512), _node_spec(128)],
        out_shape=[jax.ShapeDtypeStruct((n, 512), jnp.float32),
                   jax.ShapeDtypeStruct((n, 128), jnp.float32)],
    )(x, W_gat, att_src, att_dst)

    gather128 = _sc_gather(128)
    gs = gather128(asd, src)
    gd = gather128(asd, dst)

    ex16 = pl.pallas_call(
        _tc2_body,
        grid=(egrid,),
        in_specs=[_edge_spec(128), _edge_spec(128)],
        out_specs=_edge_spec(128),
        out_shape=jax.ShapeDtypeStruct((EPAD, 128), jnp.float32),
    )(gs, gd)

    scat128 = _sc_scatter_add(128)
    z128 = jnp.zeros((n, 128), jnp.float32)
    dd2 = scat128(ex16, dst, z128)

    q = pl.pallas_call(
        _tc4_body,
        grid=(ngrid,),
        in_specs=[_part_spec(128)],
        out_specs=_node_spec(128),
        out_shape=jax.ShapeDtypeStruct((n, 128), jnp.float32),
    )(dd2)

    qd = gather128(q, dst)
    qs = gather128(q, src)

    al = pl.pallas_call(
        _tc5_body,
        grid=(egrid,),
        in_specs=[_edge_spec(128)] * 3,
        out_specs=_edge_spec(16),
        out_shape=jax.ShapeDtypeStruct((EPAD, 16), jnp.float32),
    )(ex16, qs, qd)

    hs = _sc_gather(512)(h, src)

    msgs = pl.pallas_call(
        _tc6_body,
        grid=(egrid,),
        in_specs=[_edge_spec(512), _edge_spec(16)],
        out_specs=[_edge_spec(128)] * 4,
        out_shape=[jax.ShapeDtypeStruct((EPAD, 128), jnp.float32)] * 4,
    )(hs, al)

    parts = [scat128(m, dst, z128) for m in msgs]

    xw1 = pl.pallas_call(
        _tc7_body,
        grid=(ngrid,),
        in_specs=[_part_spec(128)] * 4 + [
            _full_spec((1, 512)), _full_spec((512, 64)), _full_spec((1, 64)),
            _full_spec((64, 64))],
        out_specs=_node_spec(128),
        out_shape=jax.ShapeDtypeStruct((n, 128), jnp.float32),
    )(*parts, b_gat.reshape(1, 512), W_emb, b_emb.reshape(1, 64), W1)

    x1s = gather128(xw1, src)
    m1 = pl.pallas_call(
        _tc8_body,
        grid=(egrid,),
        in_specs=[_edge_spec(128), _edge_spec(16)],
        out_specs=_edge_spec(128),
        out_shape=jax.ShapeDtypeStruct((EPAD, 128), jnp.float32),
    )(x1s, al)
    p1 = scat128(m1, dst, z128)

    xw2 = pl.pallas_call(
        _tc9_body,
        grid=(ngrid,),
        in_specs=[_part_spec(128), _full_spec((1, 64)), _full_spec((64, 128))],
        out_specs=_node_spec(128),
        out_shape=jax.ShapeDtypeStruct((n, 128), jnp.float32),
    )(p1, b1.reshape(1, 64), W2)

    x2s = gather128(xw2, src)
    m2 = pl.pallas_call(
        _tc8_body,
        grid=(egrid,),
        in_specs=[_edge_spec(128), _edge_spec(16)],
        out_specs=_edge_spec(128),
        out_shape=jax.ShapeDtypeStruct((EPAD, 128), jnp.float32),
    )(x2s, al)
    p2 = _sc_scatter_add(128)(m2, dst, z128)

    out = pl.pallas_call(
        _tc11_body,
        grid=(ngrid,),
        in_specs=[_part_spec(128), _full_spec((1, 128))],
        out_specs=_node_spec(128),
        out_shape=jax.ShapeDtypeStruct((n, 128), jnp.float32),
    )(p2, b2.reshape(1, 128))
    return out
